# baseline (speedup 1.0000x reference)
import functools

import jax
import jax.numpy as jnp
from jax import lax
from jax.experimental import pallas as pl
from jax.experimental.pallas import tpu as pltpu
from jax.experimental.pallas import tpu_sc as plsc

_K = 64
_SLOTS = 4


def _row_split(n, ns):
    assert n % 8 == 0
    ngrp = n // 8
    return (ngrp // ns) * 8, ngrp % ns


def _pad_chunks(e, nw):
    return -(-e // (_K * nw * 8)) * 8


def _idx_block(nchunk):
    return max(b for b in range(8, min(nchunk, 16) + 1, 8) if nchunk % b == 0)


def _make_pipeline(nchunk, nblk, b):
    assert nchunk % _SLOTS == 0 and b % _SLOTS == 0

    def pipeline(tab, acc, s2d, d2d, row0, isb, idb, rows3, *sems):
        sg = sems[:_SLOTS]
        ss = sems[_SLOTS:]

        def load_block(blk):
            slot = lax.rem(blk, 2)
            base = row0 + blk * b
            pltpu.sync_copy(s2d.at[pl.ds(base, b)], isb.at[slot])
            pltpu.sync_copy(d2d.at[pl.ds(base, b)], idb.at[slot])

        def src_at(c):
            return tab.at[isb.at[lax.rem(c // b, 2), lax.rem(c, b)]]

        def dst_at(c):
            return acc.at[idb.at[lax.rem(c // b, 2), lax.rem(c, b)]]

        def gather(c, m):
            pltpu.async_copy(src_at(c), rows3.at[m], sg[m])

        load_block(0)
        for m in range(2):
            gather(m, m)

        def group(j, carry):
            c0 = _SLOTS * j
            for m in range(_SLOTS):
                c = c0 + m

                @pl.when(jnp.logical_and(lax.rem(c, b) == b - 2,
                                         c + 2 < nchunk))
                def _(c=c):
                    load_block(c // b + 1)

                pltpu.make_async_copy(src_at(c), rows3.at[m], sg[m]).wait()
                pltpu.async_copy(rows3.at[m], dst_at(c), ss[m], add=True)

                m2 = (m + 2) % _SLOTS

                @pl.when(c + 2 < nchunk)
                def _(c=c, m2=m2):
                    if m < 2:
                        @pl.when(c >= 2)
                        def _():
                            pltpu.make_async_copy(
                                rows3.at[m2], dst_at(c - 2), ss[m2]).wait()
                    else:
                        pltpu.make_async_copy(
                            rows3.at[m2], dst_at(c - 2), ss[m2]).wait()
                    gather(c + 2, m2)

            return carry

        lax.fori_loop(0, nchunk // _SLOTS, group, 0)
        for m in range(_SLOTS):
            c = nchunk - _SLOTS + m
            pltpu.make_async_copy(rows3.at[m], dst_at(c), ss[m]).wait()

    return pipeline


_DEAD = 128


def _sc_scratch(n, d, b):
    return [
        pltpu.VMEM_SHARED((n + _DEAD, d), jnp.float32),
        pltpu.VMEM((2, b, _K), jnp.int32),
        pltpu.VMEM((2, b, _K), jnp.int32),
        pltpu.VMEM((_SLOTS, _K, d), jnp.float32),
    ] + [pltpu.SemaphoreType.DMA] * (2 * _SLOTS)


@functools.lru_cache(maxsize=None)
def _make_sc_agg_featsplit(n, e, d2):
    info = plsc.get_sparse_core_info()
    ns = info.num_subcores
    assert d2 % 128 == 0
    nchunk = _pad_chunks(e, ns)
    b = _idx_block(nchunk)
    nblk = nchunk // b
    rpw, grem = _row_split(n, ns)
    pipeline = _make_pipeline(nchunk, nblk, b)

    mesh = plsc.VectorSubcoreMesh(core_axis_name="c", subcore_axis_name="s")

    def body(t0, t1, s2d, d2d, out0, out1, acc, isb, idb, rows3, *sems):
        cid = lax.axis_index("c")
        sid = lax.axis_index("s")
        r0 = sid * rpw
        rem0 = ns * rpw + sid * 8
        row0 = sid * nchunk

        def work(tab, out):
            pltpu.sync_copy(tab.at[pl.ds(r0, rpw)], acc.at[pl.ds(r0, rpw)])
            if grem:
                @pl.when(sid < grem)
                def _():
                    pltpu.sync_copy(tab.at[pl.ds(rem0, 8)],
                                    acc.at[pl.ds(rem0, 8)])
            plsc.subcore_barrier()
            pipeline(tab, acc, s2d, d2d, row0, isb, idb, rows3, *sems)
            plsc.subcore_barrier()
            pltpu.sync_copy(acc.at[pl.ds(r0, rpw)], out.at[pl.ds(r0, rpw)])
            if grem:
                @pl.when(sid < grem)
                def _():
                    pltpu.sync_copy(acc.at[pl.ds(rem0, 8)],
                                    out.at[pl.ds(rem0, 8)])

        @pl.when(cid == 0)
        def _():
            work(t0, out0)

        @pl.when(cid == 1)
        def _():
            work(t1, out1)

    return pl.kernel(
        body,
        out_type=(jax.ShapeDtypeStruct((n, d2), jnp.float32),
                  jax.ShapeDtypeStruct((n, d2), jnp.float32)),
        mesh=mesh,
        scratch_types=_sc_scratch(n, d2, b),
    )


@functools.lru_cache(maxsize=None)
def _make_sc_agg_edgesplit(n, e, d):
    info = plsc.get_sparse_core_info()
    nc, ns = info.num_cores, info.num_subcores
    nw = nc * ns
    assert d % 128 == 0
    nchunk = _pad_chunks(e, nw)
    b = _idx_block(nchunk)
    nblk = nchunk // b
    rpw, grem = _row_split(n, ns)
    pipeline = _make_pipeline(nchunk, nblk, b)

    mesh = plsc.VectorSubcoreMesh(core_axis_name="c", subcore_axis_name="s")

    def body(tab, zer, s2d, d2d, out0, out1, acc, isb, idb, rows3, *sems):
        cid = lax.axis_index("c")
        sid = lax.axis_index("s")
        r0 = sid * rpw
        rem0 = ns * rpw + sid * 8
        row0 = (cid * ns + sid) * nchunk

        def init(seed_ref):
            pltpu.sync_copy(seed_ref.at[pl.ds(r0, rpw)], acc.at[pl.ds(r0, rpw)])
            if grem:
                @pl.when(sid < grem)
                def _():
                    pltpu.sync_copy(seed_ref.at[pl.ds(rem0, 8)],
                                    acc.at[pl.ds(rem0, 8)])

        @pl.when(cid == 0)
        def _():
            init(tab)

        @pl.when(cid == 1)
        def _():
            init(zer)

        plsc.subcore_barrier()
        pipeline(tab, acc, s2d, d2d, row0, isb, idb, rows3, *sems)
        plsc.subcore_barrier()

        def copyout(out):
            pltpu.sync_copy(acc.at[pl.ds(r0, rpw)], out.at[pl.ds(r0, rpw)])
            if grem:
                @pl.when(sid < grem)
                def _():
                    pltpu.sync_copy(acc.at[pl.ds(rem0, 8)],
                                    out.at[pl.ds(rem0, 8)])

        @pl.when(cid == 0)
        def _():
            copyout(out0)

        @pl.when(cid == 1)
        def _():
            copyout(out1)

    return pl.kernel(
        body,
        out_type=(jax.ShapeDtypeStruct((n, d), jnp.float32),
                  jax.ShapeDtypeStruct((n, d), jnp.float32)),
        mesh=mesh,
        scratch_types=_sc_scratch(n, d, b),
    )


def _bn_relu(z, g, b):
    mu = jnp.mean(z, axis=0, keepdims=True)
    var = jnp.mean((z - mu) ** 2, axis=0, keepdims=True)
    return jnp.maximum((z - mu) * lax.rsqrt(var + 1e-5) * g + b, 0.0)


@functools.lru_cache(maxsize=None)
def _make_mlp(n, d_in2, d_h, d_out, combine, split_out):
    def body(a0, a1, wa, wb, gm, bm, g, b, *outs):
        if combine == "concat":
            s = jnp.concatenate([a0[...], a1[...]], axis=1)
        else:
            s = a0[...] + a1[...]
        z = jnp.dot(s, wa[...], preferred_element_type=jnp.float32)
        z = _bn_relu(z, gm[...], bm[...])
        h = jnp.dot(z, wb[...], preferred_element_type=jnp.float32)
        h = _bn_relu(h, g[...], b[...])
        if split_out:
            outs[0][...] = h[:, : d_out // 2]
            outs[1][...] = h[:, d_out // 2:]
        else:
            outs[0][...] = h

    if split_out:
        out_shape = (jax.ShapeDtypeStruct((n, d_out // 2), jnp.float32),
                     jax.ShapeDtypeStruct((n, d_out // 2), jnp.float32))
    else:
        out_shape = jax.ShapeDtypeStruct((n, d_out), jnp.float32)
    return pl.pallas_call(body, out_shape=out_shape)


def _pad_edges(idx, e_pad, base):
    npad = e_pad - idx.shape[0]
    pad = base + jnp.arange(npad, dtype=idx.dtype) % _DEAD
    return jnp.concatenate([idx, pad]).reshape(-1, _K)


def kernel(x, edge_index, W0a, g0m, b0m, W0b, g0, b0, W1a, g1m, b1m, W1b, g1, b1):
    n, d_in = x.shape
    e = edge_index.shape[1]
    d_h = W0a.shape[1]
    d_out = W1b.shape[1]
    info = plsc.get_sparse_core_info()
    nc, ns = info.num_cores, info.num_subcores
    src = edge_index[0]
    dst = edge_index[1]

    r2 = lambda v: v.reshape(1, -1)

    e_pad0 = _pad_chunks(e, nc * ns) * _K * nc * ns
    s2d0 = _pad_edges(src, e_pad0, 0)
    d2d0 = _pad_edges(dst, e_pad0, n)
    zer = jnp.zeros_like(x)
    a0, a1 = _make_sc_agg_edgesplit(n, e, d_in)(x, zer, s2d0, d2d0)
    h0, h1 = _make_mlp(n, d_in, d_h, d_h, "add", True)(
        a0, a1, W0a, W0b, r2(g0m), r2(b0m), r2(g0), r2(b0))

    e_pad1 = _pad_chunks(e, ns) * _K * ns
    s2d1 = _pad_edges(src, e_pad1, 0)
    d2d1 = _pad_edges(dst, e_pad1, n)
    b0_, b1_ = _make_sc_agg_featsplit(n, e, d_h // 2)(h0, h1, s2d1, d2d1)
    out = _make_mlp(n, d_h // 2, d_h, d_out, "concat", False)(
        b0_, b1_, W1a, W1b, r2(g1m), r2(b1m), r2(g1), r2(b1))
    return out


# speedup vs baseline: 1.1397x; 1.1397x over previous
import functools

import jax
import jax.numpy as jnp
from jax import lax
from jax.experimental import pallas as pl
from jax.experimental.pallas import tpu as pltpu
from jax.experimental.pallas import tpu_sc as plsc

_K = 128


def _row_split(n, ns):
    assert n % 8 == 0
    ngrp = n // 8
    return (ngrp // ns) * 8, ngrp % ns


def _pad_chunks(e, nw):
    return -(-e // (_K * nw * 8)) * 8


def _idx_block(nchunk):
    return max(b for b in range(8, min(nchunk, 16) + 1, 8) if nchunk % b == 0)


def _make_pipeline(nchunk, nblk, b):
    def pipeline(tab, acc, s2d, d2d, row0, isb, idb, rows2, sg0, sg1):
        def load_block(blk):
            slot = lax.rem(blk, 2)
            base = row0 + blk * b
            pltpu.sync_copy(s2d.at[pl.ds(base, b)], isb.at[slot])
            pltpu.sync_copy(d2d.at[pl.ds(base, b)], idb.at[slot])

        def gather(c, slot, sem):
            blk = c // b
            pltpu.async_copy(
                tab.at[isb.at[lax.rem(blk, 2), lax.rem(c, b)]],
                rows2.at[slot], sem)

        def drain_scatter(c, slot, sem):
            blk = c // b
            r = lax.rem(c, b)
            pltpu.make_async_copy(
                tab.at[isb.at[lax.rem(blk, 2), r]], rows2.at[slot], sem
            ).wait()
            pltpu.sync_copy(rows2.at[slot],
                            acc.at[idb.at[lax.rem(blk, 2), r]], add=True)

        load_block(0)
        gather(0, 0, sg0)
        gather(1, 1, sg1)

        def pair(j, carry):
            c0 = 2 * j

            @pl.when(jnp.logical_and(lax.rem(c0, b) == b - 2,
                                     c0 + 2 < nchunk))
            def _():
                load_block(c0 // b + 1)

            drain_scatter(c0, 0, sg0)

            @pl.when(c0 + 2 < nchunk)
            def _():
                gather(c0 + 2, 0, sg0)

            drain_scatter(c0 + 1, 1, sg1)

            @pl.when(c0 + 3 < nchunk)
            def _():
                gather(c0 + 3, 1, sg1)

            return carry

        lax.fori_loop(0, nchunk // 2, pair, 0)

    return pipeline


_DEAD = 128


def _sc_scratch(n, d, b):
    return [
        pltpu.VMEM_SHARED((n + _DEAD, d), jnp.float32),
        pltpu.VMEM((2, b, _K), jnp.int32),
        pltpu.VMEM((2, b, _K), jnp.int32),
        pltpu.VMEM((2, _K, d), jnp.float32),
        pltpu.SemaphoreType.DMA,
        pltpu.SemaphoreType.DMA,
    ]


@functools.lru_cache(maxsize=None)
def _make_sc_agg_featsplit(n, e, d2):
    info = plsc.get_sparse_core_info()
    ns = info.num_subcores
    assert d2 % 128 == 0
    nchunk = _pad_chunks(e, ns)
    b = _idx_block(nchunk)
    nblk = nchunk // b
    rpw, grem = _row_split(n, ns)
    pipeline = _make_pipeline(nchunk, nblk, b)

    mesh = plsc.VectorSubcoreMesh(core_axis_name="c", subcore_axis_name="s")

    def body(t0, t1, s2d, d2d, out0, out1, acc, isb, idb, rows2, sg0, sg1):
        cid = lax.axis_index("c")
        sid = lax.axis_index("s")
        r0 = sid * rpw
        rem0 = ns * rpw + sid * 8
        row0 = sid * nchunk

        def work(tab, out):
            pltpu.sync_copy(tab.at[pl.ds(r0, rpw)], acc.at[pl.ds(r0, rpw)])
            if grem:
                @pl.when(sid < grem)
                def _():
                    pltpu.sync_copy(tab.at[pl.ds(rem0, 8)],
                                    acc.at[pl.ds(rem0, 8)])
            plsc.subcore_barrier()
            pipeline(tab, acc, s2d, d2d, row0, isb, idb, rows2, sg0, sg1)
            plsc.subcore_barrier()
            pltpu.sync_copy(acc.at[pl.ds(r0, rpw)], out.at[pl.ds(r0, rpw)])
            if grem:
                @pl.when(sid < grem)
                def _():
                    pltpu.sync_copy(acc.at[pl.ds(rem0, 8)],
                                    out.at[pl.ds(rem0, 8)])

        @pl.when(cid == 0)
        def _():
            work(t0, out0)

        @pl.when(cid == 1)
        def _():
            work(t1, out1)

    return pl.kernel(
        body,
        out_type=(jax.ShapeDtypeStruct((n, d2), jnp.float32),
                  jax.ShapeDtypeStruct((n, d2), jnp.float32)),
        mesh=mesh,
        scratch_types=_sc_scratch(n, d2, b),
    )


@functools.lru_cache(maxsize=None)
def _make_sc_agg_edgesplit(n, e, d):
    info = plsc.get_sparse_core_info()
    nc, ns = info.num_cores, info.num_subcores
    nw = nc * ns
    assert d % 128 == 0
    nchunk = _pad_chunks(e, nw)
    b = _idx_block(nchunk)
    nblk = nchunk // b
    rpw, grem = _row_split(n, ns)
    pipeline = _make_pipeline(nchunk, nblk, b)

    mesh = plsc.VectorSubcoreMesh(core_axis_name="c", subcore_axis_name="s")

    def body(tab, s2d, d2d, out0, out1, acc, isb, idb, rows2, sg0, sg1):
        cid = lax.axis_index("c")
        sid = lax.axis_index("s")
        r0 = sid * rpw
        rem0 = ns * rpw + sid * 8
        row0 = (cid * ns + sid) * nchunk

        pltpu.sync_copy(tab.at[pl.ds(r0, rpw)], acc.at[pl.ds(r0, rpw)])
        if grem:
            @pl.when(sid < grem)
            def _():
                pltpu.sync_copy(tab.at[pl.ds(rem0, 8)],
                                acc.at[pl.ds(rem0, 8)])

        plsc.subcore_barrier()
        pipeline(tab, acc, s2d, d2d, row0, isb, idb, rows2, sg0, sg1)
        plsc.subcore_barrier()

        def copyout(out):
            pltpu.sync_copy(acc.at[pl.ds(r0, rpw)], out.at[pl.ds(r0, rpw)])
            if grem:
                @pl.when(sid < grem)
                def _():
                    pltpu.sync_copy(acc.at[pl.ds(rem0, 8)],
                                    out.at[pl.ds(rem0, 8)])

        @pl.when(cid == 0)
        def _():
            copyout(out0)

        @pl.when(cid == 1)
        def _():
            copyout(out1)

    return pl.kernel(
        body,
        out_type=(jax.ShapeDtypeStruct((n, d), jnp.float32),
                  jax.ShapeDtypeStruct((n, d), jnp.float32)),
        mesh=mesh,
        scratch_types=_sc_scratch(n, d, b),
    )


def _bn_relu(z, g, b):
    mu = jnp.mean(z, axis=0, keepdims=True)
    var = jnp.mean((z - mu) ** 2, axis=0, keepdims=True)
    return jnp.maximum((z - mu) * lax.rsqrt(var + 1e-5) * g + b, 0.0)


@functools.lru_cache(maxsize=None)
def _make_mlp(n, d_in2, d_h, d_out, combine, split_out):
    def body(*refs):
        if combine == "concat":
            a0, a1, wa, wb, gm, bm, g, b, *outs = refs
            s = jnp.concatenate([a0[...], a1[...]], axis=1)
        else:
            a0, a1, hin, wa, wb, gm, bm, g, b, *outs = refs
            s = a0[...] + a1[...] - hin[...]
        z = jnp.dot(s, wa[...], preferred_element_type=jnp.float32)
        z = _bn_relu(z, gm[...], bm[...])
        h = jnp.dot(z, wb[...], preferred_element_type=jnp.float32)
        h = _bn_relu(h, g[...], b[...])
        if split_out:
            outs[0][...] = h[:, : d_out // 2]
            outs[1][...] = h[:, d_out // 2:]
        else:
            outs[0][...] = h

    if split_out:
        out_shape = (jax.ShapeDtypeStruct((n, d_out // 2), jnp.float32),
                     jax.ShapeDtypeStruct((n, d_out // 2), jnp.float32))
    else:
        out_shape = jax.ShapeDtypeStruct((n, d_out), jnp.float32)
    return pl.pallas_call(body, out_shape=out_shape)


def _pad_edges(idx, e_pad, base):
    npad = e_pad - idx.shape[0]
    pad = base + jnp.arange(npad, dtype=idx.dtype) % _DEAD
    return jnp.concatenate([idx, pad]).reshape(-1, _K)


def kernel(x, edge_index, W0a, g0m, b0m, W0b, g0, b0, W1a, g1m, b1m, W1b, g1, b1):
    n, d_in = x.shape
    e = edge_index.shape[1]
    d_h = W0a.shape[1]
    d_out = W1b.shape[1]
    info = plsc.get_sparse_core_info()
    nc, ns = info.num_cores, info.num_subcores
    src = edge_index[0]
    dst = edge_index[1]

    r2 = lambda v: v.reshape(1, -1)

    e_pad0 = _pad_chunks(e, nc * ns) * _K * nc * ns
    s2d0 = _pad_edges(src, e_pad0, 0)
    d2d0 = _pad_edges(dst, e_pad0, n)
    a0, a1 = _make_sc_agg_edgesplit(n, e, d_in)(x, s2d0, d2d0)
    h0, h1 = _make_mlp(n, d_in, d_h, d_h, "add", True)(
        a0, a1, x, W0a, W0b, r2(g0m), r2(b0m), r2(g0), r2(b0))

    e_pad1 = _pad_chunks(e, ns) * _K * ns
    s2d1 = _pad_edges(src, e_pad1, 0)
    d2d1 = _pad_edges(dst, e_pad1, n)
    b0_, b1_ = _make_sc_agg_featsplit(n, e, d_h // 2)(h0, h1, s2d1, d2d1)
    out = _make_mlp(n, d_h // 2, d_h, d_out, "concat", False)(
        b0_, b1_, W1a, W1b, r2(g1m), r2(b1m), r2(g1), r2(b1))
    return out


# speedup vs baseline: 1.1693x; 1.0260x over previous
import functools

import jax
import jax.numpy as jnp
from jax import lax
from jax.experimental import pallas as pl
from jax.experimental.pallas import tpu as pltpu
from jax.experimental.pallas import tpu_sc as plsc

_K = 128


def _row_split(n, ns):
    assert n % 8 == 0
    ngrp = n // 8
    return (ngrp // ns) * 8, ngrp % ns


def _pad_chunks(e, nw):
    return -(-e // (_K * nw * 8)) * 8


def _idx_block(nchunk):
    return max(b for b in range(8, min(nchunk, 16) + 1, 8) if nchunk % b == 0)


def _make_pipeline(nchunk, nblk, b):
    def pipeline(tab, acc, s2d, d2d, row0, isb, idb, rows2, sg0, sg1, si):
        def load_block(blk):
            slot = lax.rem(blk, 2)
            base = row0 + blk * b
            pltpu.sync_copy(s2d.at[pl.ds(base, b)], isb.at[slot])
            pltpu.sync_copy(d2d.at[pl.ds(base, b)], idb.at[slot])

        def start_load_block(blk):
            slot = lax.rem(blk, 2)
            base = row0 + blk * b
            pltpu.async_copy(s2d.at[pl.ds(base, b)], isb.at[slot], si)
            pltpu.async_copy(d2d.at[pl.ds(base, b)], idb.at[slot], si)

        def wait_load_block(blk):
            slot = lax.rem(blk, 2)
            base = row0 + blk * b
            pltpu.make_async_copy(s2d.at[pl.ds(base, b)], isb.at[slot],
                                  si).wait()
            pltpu.make_async_copy(d2d.at[pl.ds(base, b)], idb.at[slot],
                                  si).wait()

        def gather(c, slot, sem):
            blk = c // b
            pltpu.async_copy(
                tab.at[isb.at[lax.rem(blk, 2), lax.rem(c, b)]],
                rows2.at[slot], sem)

        def drain_scatter(c, slot, sem):
            blk = c // b
            r = lax.rem(c, b)
            pltpu.make_async_copy(
                tab.at[isb.at[lax.rem(blk, 2), r]], rows2.at[slot], sem
            ).wait()
            pltpu.sync_copy(rows2.at[slot],
                            acc.at[idb.at[lax.rem(blk, 2), r]], add=True)

        load_block(0)
        gather(0, 0, sg0)
        gather(1, 1, sg1)

        def pair(j, carry):
            c0 = 2 * j

            @pl.when(jnp.logical_and(lax.rem(c0, b) == b - 4,
                                     c0 + 4 < nchunk))
            def _():
                start_load_block(c0 // b + 1)

            @pl.when(jnp.logical_and(lax.rem(c0, b) == b - 2,
                                     c0 + 2 < nchunk))
            def _():
                wait_load_block(c0 // b + 1)

            drain_scatter(c0, 0, sg0)

            @pl.when(c0 + 2 < nchunk)
            def _():
                gather(c0 + 2, 0, sg0)

            drain_scatter(c0 + 1, 1, sg1)

            @pl.when(c0 + 3 < nchunk)
            def _():
                gather(c0 + 3, 1, sg1)

            return carry

        lax.fori_loop(0, nchunk // 2, pair, 0)

    return pipeline


_DEAD = 128


def _sc_scratch(n, d, b):
    return [
        pltpu.VMEM_SHARED((n + _DEAD, d), jnp.float32),
        pltpu.VMEM((2, b, _K), jnp.int32),
        pltpu.VMEM((2, b, _K), jnp.int32),
        pltpu.VMEM((2, _K, d), jnp.float32),
        pltpu.SemaphoreType.DMA,
        pltpu.SemaphoreType.DMA,
        pltpu.SemaphoreType.DMA,
    ]


@functools.lru_cache(maxsize=None)
def _make_sc_agg_featsplit(n, e, d2):
    info = plsc.get_sparse_core_info()
    ns = info.num_subcores
    assert d2 % 128 == 0
    nchunk = _pad_chunks(e, ns)
    b = _idx_block(nchunk)
    nblk = nchunk // b
    rpw, grem = _row_split(n, ns)
    pipeline = _make_pipeline(nchunk, nblk, b)

    mesh = plsc.VectorSubcoreMesh(core_axis_name="c", subcore_axis_name="s")

    def body(t0, t1, s2d, d2d, out0, out1, acc, isb, idb, rows2, sg0, sg1, si):
        cid = lax.axis_index("c")
        sid = lax.axis_index("s")
        r0 = sid * rpw
        rem0 = ns * rpw + sid * 8
        row0 = sid * nchunk

        def work(tab, out):
            pltpu.sync_copy(tab.at[pl.ds(r0, rpw)], acc.at[pl.ds(r0, rpw)])
            if grem:
                @pl.when(sid < grem)
                def _():
                    pltpu.sync_copy(tab.at[pl.ds(rem0, 8)],
                                    acc.at[pl.ds(rem0, 8)])
            plsc.subcore_barrier()
            pipeline(tab, acc, s2d, d2d, row0, isb, idb, rows2, sg0, sg1, si)
            plsc.subcore_barrier()
            pltpu.sync_copy(acc.at[pl.ds(r0, rpw)], out.at[pl.ds(r0, rpw)])
            if grem:
                @pl.when(sid < grem)
                def _():
                    pltpu.sync_copy(acc.at[pl.ds(rem0, 8)],
                                    out.at[pl.ds(rem0, 8)])

        @pl.when(cid == 0)
        def _():
            work(t0, out0)

        @pl.when(cid == 1)
        def _():
            work(t1, out1)

    return pl.kernel(
        body,
        out_type=(jax.ShapeDtypeStruct((n, d2), jnp.float32),
                  jax.ShapeDtypeStruct((n, d2), jnp.float32)),
        mesh=mesh,
        scratch_types=_sc_scratch(n, d2, b),
    )


@functools.lru_cache(maxsize=None)
def _make_sc_agg_edgesplit(n, e, d):
    info = plsc.get_sparse_core_info()
    nc, ns = info.num_cores, info.num_subcores
    nw = nc * ns
    assert d % 128 == 0
    nchunk = _pad_chunks(e, nw)
    b = _idx_block(nchunk)
    nblk = nchunk // b
    rpw, grem = _row_split(n, ns)
    pipeline = _make_pipeline(nchunk, nblk, b)

    mesh = plsc.VectorSubcoreMesh(core_axis_name="c", subcore_axis_name="s")

    def body(tab, s2d, d2d, out0, out1, acc, isb, idb, rows2, sg0, sg1, si):
        cid = lax.axis_index("c")
        sid = lax.axis_index("s")
        r0 = sid * rpw
        rem0 = ns * rpw + sid * 8
        row0 = (cid * ns + sid) * nchunk

        pltpu.sync_copy(tab.at[pl.ds(r0, rpw)], acc.at[pl.ds(r0, rpw)])
        if grem:
            @pl.when(sid < grem)
            def _():
                pltpu.sync_copy(tab.at[pl.ds(rem0, 8)],
                                acc.at[pl.ds(rem0, 8)])

        plsc.subcore_barrier()
        pipeline(tab, acc, s2d, d2d, row0, isb, idb, rows2, sg0, sg1, si)
        plsc.subcore_barrier()

        def copyout(out):
            pltpu.sync_copy(acc.at[pl.ds(r0, rpw)], out.at[pl.ds(r0, rpw)])
            if grem:
                @pl.when(sid < grem)
                def _():
                    pltpu.sync_copy(acc.at[pl.ds(rem0, 8)],
                                    out.at[pl.ds(rem0, 8)])

        @pl.when(cid == 0)
        def _():
            copyout(out0)

        @pl.when(cid == 1)
        def _():
            copyout(out1)

    return pl.kernel(
        body,
        out_type=(jax.ShapeDtypeStruct((n, d), jnp.float32),
                  jax.ShapeDtypeStruct((n, d), jnp.float32)),
        mesh=mesh,
        scratch_types=_sc_scratch(n, d, b),
    )


def _bn_relu(z, g, b):
    mu = jnp.mean(z, axis=0, keepdims=True)
    var = jnp.mean((z - mu) ** 2, axis=0, keepdims=True)
    return jnp.maximum((z - mu) * lax.rsqrt(var + 1e-5) * g + b, 0.0)


@functools.lru_cache(maxsize=None)
def _make_mlp(n, d_in2, d_h, d_out, combine, split_out):
    def body(*refs):
        if combine == "concat":
            a0, a1, wa, wb, gm, bm, g, b, *outs = refs
            s = jnp.concatenate([a0[...], a1[...]], axis=1)
        else:
            a0, a1, hin, wa, wb, gm, bm, g, b, *outs = refs
            s = a0[...] + a1[...] - hin[...]
        z = jnp.dot(s, wa[...], preferred_element_type=jnp.float32)
        z = _bn_relu(z, gm[...], bm[...])
        h = jnp.dot(z, wb[...], preferred_element_type=jnp.float32)
        h = _bn_relu(h, g[...], b[...])
        if split_out:
            outs[0][...] = h[:, : d_out // 2]
            outs[1][...] = h[:, d_out // 2:]
        else:
            outs[0][...] = h

    if split_out:
        out_shape = (jax.ShapeDtypeStruct((n, d_out // 2), jnp.float32),
                     jax.ShapeDtypeStruct((n, d_out // 2), jnp.float32))
    else:
        out_shape = jax.ShapeDtypeStruct((n, d_out), jnp.float32)
    return pl.pallas_call(body, out_shape=out_shape)


def _pad_edges(idx, e_pad, base):
    npad = e_pad - idx.shape[0]
    pad = base + jnp.arange(npad, dtype=idx.dtype) % _DEAD
    return jnp.concatenate([idx, pad]).reshape(-1, _K)


def kernel(x, edge_index, W0a, g0m, b0m, W0b, g0, b0, W1a, g1m, b1m, W1b, g1, b1):
    n, d_in = x.shape
    e = edge_index.shape[1]
    d_h = W0a.shape[1]
    d_out = W1b.shape[1]
    info = plsc.get_sparse_core_info()
    nc, ns = info.num_cores, info.num_subcores
    src = edge_index[0]
    dst = edge_index[1]

    r2 = lambda v: v.reshape(1, -1)

    e_pad0 = _pad_chunks(e, nc * ns) * _K * nc * ns
    s2d0 = _pad_edges(src, e_pad0, 0)
    d2d0 = _pad_edges(dst, e_pad0, n)
    a0, a1 = _make_sc_agg_edgesplit(n, e, d_in)(x, s2d0, d2d0)
    h0, h1 = _make_mlp(n, d_in, d_h, d_h, "add", True)(
        a0, a1, x, W0a, W0b, r2(g0m), r2(b0m), r2(g0), r2(b0))

    e_pad1 = _pad_chunks(e, ns) * _K * ns
    s2d1 = _pad_edges(src, e_pad1, 0)
    d2d1 = _pad_edges(dst, e_pad1, n)
    b0_, b1_ = _make_sc_agg_featsplit(n, e, d_h // 2)(h0, h1, s2d1, d2d1)
    out = _make_mlp(n, d_h // 2, d_h, d_out, "concat", False)(
        b0_, b1_, W1a, W1b, r2(g1m), r2(b1m), r2(g1), r2(b1))
    return out


# speedup vs baseline: 1.1698x; 1.0005x over previous
import functools

import jax
import jax.numpy as jnp
from jax import lax
from jax.experimental import pallas as pl
from jax.experimental.pallas import tpu as pltpu
from jax.experimental.pallas import tpu_sc as plsc

_K = 128


def _row_split(n, ns):
    assert n % 8 == 0
    ngrp = n // 8
    return (ngrp // ns) * 8, ngrp % ns


def _pad_chunks(e, nw):
    return -(-e // (_K * nw * 8)) * 8


def _idx_block(nchunk):
    return max(b for b in range(8, min(nchunk, 32) + 1, 8) if nchunk % b == 0)


def _make_pipeline(nchunk, nblk, b):
    def pipeline(tab, acc, s2d, d2d, row0, isb, idb, rows2, sg0, sg1, si):
        def load_block(blk):
            slot = lax.rem(blk, 2)
            base = row0 + blk * b
            pltpu.sync_copy(s2d.at[pl.ds(base, b)], isb.at[slot])
            pltpu.sync_copy(d2d.at[pl.ds(base, b)], idb.at[slot])

        def start_load_block(blk):
            slot = lax.rem(blk, 2)
            base = row0 + blk * b
            pltpu.async_copy(s2d.at[pl.ds(base, b)], isb.at[slot], si)
            pltpu.async_copy(d2d.at[pl.ds(base, b)], idb.at[slot], si)

        def wait_load_block(blk):
            slot = lax.rem(blk, 2)
            base = row0 + blk * b
            pltpu.make_async_copy(s2d.at[pl.ds(base, b)], isb.at[slot],
                                  si).wait()
            pltpu.make_async_copy(d2d.at[pl.ds(base, b)], idb.at[slot],
                                  si).wait()

        def gather(c, slot, sem):
            blk = c // b
            pltpu.async_copy(
                tab.at[isb.at[lax.rem(blk, 2), lax.rem(c, b)]],
                rows2.at[slot], sem)

        def drain_scatter(c, slot, sem):
            blk = c // b
            r = lax.rem(c, b)
            pltpu.make_async_copy(
                tab.at[isb.at[lax.rem(blk, 2), r]], rows2.at[slot], sem
            ).wait()
            pltpu.sync_copy(rows2.at[slot],
                            acc.at[idb.at[lax.rem(blk, 2), r]], add=True)

        load_block(0)
        gather(0, 0, sg0)
        gather(1, 1, sg1)

        def pair(j, carry):
            c0 = 2 * j

            @pl.when(jnp.logical_and(lax.rem(c0, b) == b - 4,
                                     c0 + 4 < nchunk))
            def _():
                start_load_block(c0 // b + 1)

            @pl.when(jnp.logical_and(lax.rem(c0, b) == b - 2,
                                     c0 + 2 < nchunk))
            def _():
                wait_load_block(c0 // b + 1)

            drain_scatter(c0, 0, sg0)

            @pl.when(c0 + 2 < nchunk)
            def _():
                gather(c0 + 2, 0, sg0)

            drain_scatter(c0 + 1, 1, sg1)

            @pl.when(c0 + 3 < nchunk)
            def _():
                gather(c0 + 3, 1, sg1)

            return carry

        lax.fori_loop(0, nchunk // 2, pair, 0)

    return pipeline


_DEAD = 128


def _sc_scratch(n, d, b):
    return [
        pltpu.VMEM_SHARED((n + _DEAD, d), jnp.float32),
        pltpu.VMEM((2, b, _K), jnp.int32),
        pltpu.VMEM((2, b, _K), jnp.int32),
        pltpu.VMEM((2, _K, d), jnp.float32),
        pltpu.SemaphoreType.DMA,
        pltpu.SemaphoreType.DMA,
        pltpu.SemaphoreType.DMA,
    ]


@functools.lru_cache(maxsize=None)
def _make_sc_agg_featsplit(n, e, d2):
    info = plsc.get_sparse_core_info()
    ns = info.num_subcores
    assert d2 % 128 == 0
    nchunk = _pad_chunks(e, ns)
    b = _idx_block(nchunk)
    nblk = nchunk // b
    rpw, grem = _row_split(n, ns)
    pipeline = _make_pipeline(nchunk, nblk, b)

    mesh = plsc.VectorSubcoreMesh(core_axis_name="c", subcore_axis_name="s")

    def body(t0, t1, s2d, d2d, out0, out1, acc, isb, idb, rows2, sg0, sg1, si):
        cid = lax.axis_index("c")
        sid = lax.axis_index("s")
        r0 = sid * rpw
        rem0 = ns * rpw + sid * 8
        row0 = sid * nchunk

        def work(tab, out):
            pltpu.sync_copy(tab.at[pl.ds(r0, rpw)], acc.at[pl.ds(r0, rpw)])
            if grem:
                @pl.when(sid < grem)
                def _():
                    pltpu.sync_copy(tab.at[pl.ds(rem0, 8)],
                                    acc.at[pl.ds(rem0, 8)])
            plsc.subcore_barrier()
            pipeline(tab, acc, s2d, d2d, row0, isb, idb, rows2, sg0, sg1, si)
            plsc.subcore_barrier()
            pltpu.sync_copy(acc.at[pl.ds(r0, rpw)], out.at[pl.ds(r0, rpw)])
            if grem:
                @pl.when(sid < grem)
                def _():
                    pltpu.sync_copy(acc.at[pl.ds(rem0, 8)],
                                    out.at[pl.ds(rem0, 8)])

        @pl.when(cid == 0)
        def _():
            work(t0, out0)

        @pl.when(cid == 1)
        def _():
            work(t1, out1)

    return pl.kernel(
        body,
        out_type=(jax.ShapeDtypeStruct((n, d2), jnp.float32),
                  jax.ShapeDtypeStruct((n, d2), jnp.float32)),
        mesh=mesh,
        scratch_types=_sc_scratch(n, d2, b),
    )


@functools.lru_cache(maxsize=None)
def _make_sc_agg_edgesplit(n, e, d):
    info = plsc.get_sparse_core_info()
    nc, ns = info.num_cores, info.num_subcores
    nw = nc * ns
    assert d % 128 == 0
    nchunk = _pad_chunks(e, nw)
    b = _idx_block(nchunk)
    nblk = nchunk // b
    rpw, grem = _row_split(n, ns)
    pipeline = _make_pipeline(nchunk, nblk, b)

    mesh = plsc.VectorSubcoreMesh(core_axis_name="c", subcore_axis_name="s")

    def body(tab, s2d, d2d, out0, out1, acc, isb, idb, rows2, sg0, sg1, si):
        cid = lax.axis_index("c")
        sid = lax.axis_index("s")
        r0 = sid * rpw
        rem0 = ns * rpw + sid * 8
        row0 = (cid * ns + sid) * nchunk

        pltpu.sync_copy(tab.at[pl.ds(r0, rpw)], acc.at[pl.ds(r0, rpw)])
        if grem:
            @pl.when(sid < grem)
            def _():
                pltpu.sync_copy(tab.at[pl.ds(rem0, 8)],
                                acc.at[pl.ds(rem0, 8)])

        plsc.subcore_barrier()
        pipeline(tab, acc, s2d, d2d, row0, isb, idb, rows2, sg0, sg1, si)
        plsc.subcore_barrier()

        def copyout(out):
            pltpu.sync_copy(acc.at[pl.ds(r0, rpw)], out.at[pl.ds(r0, rpw)])
            if grem:
                @pl.when(sid < grem)
                def _():
                    pltpu.sync_copy(acc.at[pl.ds(rem0, 8)],
                                    out.at[pl.ds(rem0, 8)])

        @pl.when(cid == 0)
        def _():
            copyout(out0)

        @pl.when(cid == 1)
        def _():
            copyout(out1)

    return pl.kernel(
        body,
        out_type=(jax.ShapeDtypeStruct((n, d), jnp.float32),
                  jax.ShapeDtypeStruct((n, d), jnp.float32)),
        mesh=mesh,
        scratch_types=_sc_scratch(n, d, b),
    )


def _bn_relu(z, g, b):
    mu = jnp.mean(z, axis=0, keepdims=True)
    var = jnp.mean((z - mu) ** 2, axis=0, keepdims=True)
    return jnp.maximum((z - mu) * lax.rsqrt(var + 1e-5) * g + b, 0.0)


@functools.lru_cache(maxsize=None)
def _make_mlp(n, d_in2, d_h, d_out, combine, split_out):
    def body(*refs):
        if combine == "concat":
            a0, a1, wa, wb, gm, bm, g, b, *outs = refs
            s = jnp.concatenate([a0[...], a1[...]], axis=1)
        else:
            a0, a1, hin, wa, wb, gm, bm, g, b, *outs = refs
            s = a0[...] + a1[...] - hin[...]
        z = jnp.dot(s, wa[...], preferred_element_type=jnp.float32)
        z = _bn_relu(z, gm[...], bm[...])
        h = jnp.dot(z, wb[...], preferred_element_type=jnp.float32)
        h = _bn_relu(h, g[...], b[...])
        if split_out:
            outs[0][...] = h[:, : d_out // 2]
            outs[1][...] = h[:, d_out // 2:]
        else:
            outs[0][...] = h

    if split_out:
        out_shape = (jax.ShapeDtypeStruct((n, d_out // 2), jnp.float32),
                     jax.ShapeDtypeStruct((n, d_out // 2), jnp.float32))
    else:
        out_shape = jax.ShapeDtypeStruct((n, d_out), jnp.float32)
    return pl.pallas_call(body, out_shape=out_shape)


def _pad_edges(idx, e_pad, base):
    npad = e_pad - idx.shape[0]
    pad = base + jnp.arange(npad, dtype=idx.dtype) % _DEAD
    return jnp.concatenate([idx, pad]).reshape(-1, _K)


def kernel(x, edge_index, W0a, g0m, b0m, W0b, g0, b0, W1a, g1m, b1m, W1b, g1, b1):
    n, d_in = x.shape
    e = edge_index.shape[1]
    d_h = W0a.shape[1]
    d_out = W1b.shape[1]
    info = plsc.get_sparse_core_info()
    nc, ns = info.num_cores, info.num_subcores
    src = edge_index[0]
    dst = edge_index[1]

    r2 = lambda v: v.reshape(1, -1)

    e_pad0 = _pad_chunks(e, nc * ns) * _K * nc * ns
    s2d0 = _pad_edges(src, e_pad0, 0)
    d2d0 = _pad_edges(dst, e_pad0, n)
    a0, a1 = _make_sc_agg_edgesplit(n, e, d_in)(x, s2d0, d2d0)
    h0, h1 = _make_mlp(n, d_in, d_h, d_h, "add", True)(
        a0, a1, x, W0a, W0b, r2(g0m), r2(b0m), r2(g0), r2(b0))

    e_pad1 = _pad_chunks(e, ns) * _K * ns
    s2d1 = _pad_edges(src, e_pad1, 0)
    d2d1 = _pad_edges(dst, e_pad1, n)
    b0_, b1_ = _make_sc_agg_featsplit(n, e, d_h // 2)(h0, h1, s2d1, d2d1)
    out = _make_mlp(n, d_h // 2, d_h, d_out, "concat", False)(
        b0_, b1_, W1a, W1b, r2(g1m), r2(b1m), r2(g1), r2(b1))
    return out
